# baseline (speedup 1.0000x reference)
import functools

import jax
import jax.numpy as jnp
from jax import lax
from jax.experimental import pallas as pl
from jax.experimental.pallas import tpu as pltpu
from jax.experimental.pallas import tpu_sc as plsc

BATCH = 16384
EMB_K = 16
NC = 2
NS = 16
NW = NC * NS
BPW = BATCH // NW
CHUNK = 128
NCHUNK = BPW // CHUNK
JPC = CHUNK // 16


def _mf_body(xf_hbm, w_hbm, h_hbm, out_hbm, emb_hbm,
             xv, uidx, vidx, du, dv, urows, vrows, outv, gsem, ssem):
    wid = lax.axis_index("s") * NC + lax.axis_index("c")
    base = wid * BPW

    pltpu.sync_copy(xf_hbm.at[pl.ds(wid * 8, 8), :], xv)

    iota = lax.iota(jnp.int32, 16)
    iota2 = iota * 2
    base2 = base * 2

    gathers = []
    for t in range(NCHUNK):
        for jj in range(JPC):
            j = t * JPC + jj
            off = jj * 16
            row = jnp.full((16,), j // 4, jnp.int32)
            col = iota2 + 32 * (j % 4)
            uidx[t, pl.ds(off, 16)] = plsc.load_gather(xv, [row, col])
            vidx[t, pl.ds(off, 16)] = plsc.load_gather(xv, [row, col + 1])
            drow = (base2 + 32 * j) + iota2
            du[t, pl.ds(off, 16)] = drow
            dv[t, pl.ds(off, 16)] = drow + 1
        dst = pl.ds(t * CHUNK, CHUNK)
        gathers.append(pltpu.async_copy(w_hbm.at[uidx.at[t]], urows.at[dst], gsem))
        gathers.append(pltpu.async_copy(h_hbm.at[vidx.at[t]], vrows.at[dst], gsem))
    for cp in gathers:
        cp.wait()

    scatters = []
    for t in range(NCHUNK):
        src = pl.ds(t * CHUNK, CHUNK)
        scatters.append(pltpu.async_copy(urows.at[src], emb_hbm.at[du.at[t]], ssem))
        scatters.append(pltpu.async_copy(vrows.at[src], emb_hbm.at[dv.at[t]], ssem))

    zf = jnp.zeros((16,), jnp.float32)

    def g_body(g, carry):
        rows = iota + g * 16
        acc = zf
        for k in range(EMB_K):
            ck = jnp.full((16,), k, jnp.int32)
            uu = plsc.load_gather(urows, [rows, ck])
            vv = plsc.load_gather(vrows, [rows, ck])
            acc = acc + uu * vv
        outv[g] = acc
        return carry

    lax.fori_loop(0, BPW // 16, g_body, 0)

    pltpu.sync_copy(outv, out_hbm.at[pl.ds(wid * (BPW // 16), BPW // 16), :])
    for cp in scatters:
        cp.wait()


@jax.jit
def _mf(x, W, H):
    mesh = plsc.VectorSubcoreMesh(core_axis_name="c", subcore_axis_name="s")
    f = functools.partial(
        pl.kernel,
        mesh=mesh,
        compiler_params=pltpu.CompilerParams(
            needs_layout_passes=False, use_tc_tiling_on_sc=False),
        out_type=(
            jax.ShapeDtypeStruct((BATCH // 16, 16), jnp.float32),
            jax.ShapeDtypeStruct((2 * BATCH, EMB_K), jnp.float32),
        ),
        scratch_types=[
            pltpu.VMEM((8, 128), jnp.int32),
            pltpu.VMEM((NCHUNK, CHUNK), jnp.int32),
            pltpu.VMEM((NCHUNK, CHUNK), jnp.int32),
            pltpu.VMEM((NCHUNK, CHUNK), jnp.int32),
            pltpu.VMEM((NCHUNK, CHUNK), jnp.int32),
            pltpu.VMEM((BPW, EMB_K), jnp.float32),
            pltpu.VMEM((BPW, EMB_K), jnp.float32),
            pltpu.VMEM((BPW // 16, 16), jnp.float32),
            pltpu.SemaphoreType.DMA,
            pltpu.SemaphoreType.DMA,
        ],
    )(_mf_body)
    return f(x.reshape(2 * BATCH // 128, 128), W, H)


def kernel(x, W, H):
    out2, emb2 = _mf(x, W, H)
    return (out2.reshape(BATCH), emb2.reshape(BATCH, 2 * EMB_K))
